# baseline (speedup 1.0000x reference)
import dataclasses
import functools

import jax
import jax.numpy as jnp
from jax import lax
from jax.experimental import pallas as pl
from jax.experimental.pallas import tpu as pltpu
from jax.experimental.pallas import tpu_sc as plsc

N = 10000
D = 128
SEQ = 8
NC = 2
NS = 16
NW = NC * NS
K = 128
NP = 10240
SL = NP // NS

_mesh = plsc.VectorSubcoreMesh(core_axis_name="c", subcore_axis_name="s")

_sc_params = pltpu.CompilerParams()
if "needs_layout_passes" in pltpu.CompilerParams.__dataclass_fields__:
    _sc_params = dataclasses.replace(_sc_params, needs_layout_passes=False)


def _deg_body(dst_hbm, w_hbm, zn_hbm, deg_hbm, dstb, wb, degs):
    cid = lax.axis_index("c")
    sid = lax.axis_index("s")
    wid = cid * NS + sid
    nchunk = dst_hbm.shape[1]
    pltpu.sync_copy(dst_hbm.at[wid], dstb)
    pltpu.sync_copy(w_hbm.at[wid], wb)
    pltpu.sync_copy(zn_hbm.at[pl.ds(sid * SL, SL)], degs.at[pl.ds(sid * SL, SL)])
    plsc.subcore_barrier()

    @pl.loop(0, nchunk)
    def _(c):
        pltpu.sync_copy(wb.at[c], degs.at[dstb.at[c]], add=True)

    plsc.subcore_barrier()
    pltpu.sync_copy(degs.at[pl.ds(sid * SL, SL)],
                    deg_hbm.at[cid, pl.ds(sid * SL, SL)])


def _sc_degree(dst3, w3, nchunk):
    kern = functools.partial(
        pl.kernel,
        out_type=jax.ShapeDtypeStruct((NC, NP), jnp.float32),
        mesh=_mesh,
        scratch_types=[
            pltpu.VMEM((nchunk, K), jnp.int32),
            pltpu.VMEM((nchunk, K), jnp.float32),
            pltpu.VMEM_SHARED((NP,), jnp.float32),
        ],
    )(_deg_body)
    zn = jnp.zeros((NP,), jnp.float32)
    return kern(dst3, w3, zn)


_RING = 8


def _scale_chunk(rows, wrb, m):
    @pl.loop(0, K)
    def _(i):
        wv = plsc.load_gather(
            wrb, [jnp.full((16,), m, jnp.int32), jnp.full((16,), i, jnp.int32)])
        for j in range(D // 16):
            sl = (i, pl.ds(j * 16, 16))
            rows[sl] = rows[sl] * wv


def _agg_body(srcf_hbm, dstf_hbm, wf_hbm, g_hbm, znd_hbm, out_hbm,
              sb, db, wrb, rows_a, rows_b, accs,
              esem, gsem_a, gsem_b, ssem_a, ssem_b):
    cid = lax.axis_index("c")
    sid = lax.axis_index("s")
    wid = cid * NS + sid
    nchunk = srcf_hbm.shape[0] // NW
    base = wid * nchunk
    pltpu.sync_copy(znd_hbm.at[pl.ds(sid * SL, SL)],
                    accs.at[pl.ds(sid * SL, SL)])
    for m in (0, 1):
        pltpu.sync_copy(srcf_hbm.at[base + m], sb.at[m])
        pltpu.sync_copy(dstf_hbm.at[base + m], db.at[m])
        pltpu.sync_copy(wf_hbm.at[base + m], wrb.at[m])
    plsc.subcore_barrier()

    pltpu.async_copy(g_hbm.at[sb.at[0]], rows_a, gsem_a)

    def _stage(c, m):
        pltpu.async_copy(srcf_hbm.at[base + c], sb.at[m], esem)
        pltpu.async_copy(dstf_hbm.at[base + c], db.at[m], esem)
        pltpu.async_copy(wf_hbm.at[base + c], wrb.at[m], esem)

    def _stage_wait(c, m):
        pltpu.make_async_copy(srcf_hbm.at[base + c], sb.at[m], esem).wait()
        pltpu.make_async_copy(dstf_hbm.at[base + c], db.at[m], esem).wait()
        pltpu.make_async_copy(wf_hbm.at[base + c], wrb.at[m], esem).wait()

    @pl.loop(0, nchunk, step=2)
    def _(c):
        m0 = lax.rem(c, _RING)
        m1 = lax.rem(c + 1, _RING)
        m2 = lax.rem(c + 2, _RING)
        m3 = lax.rem(c + 3, _RING)

        pltpu.make_async_copy(g_hbm.at[sb.at[m0]], rows_a, gsem_a).wait()

        @pl.when(c > 0)
        def _():
            pltpu.make_async_copy(rows_b, accs.at[db.at[m1]], ssem_b).wait()

        pltpu.async_copy(g_hbm.at[sb.at[m1]], rows_b, gsem_b)
        _scale_chunk(rows_a, wrb, m0)
        pltpu.async_copy(rows_a, accs.at[db.at[m0]], ssem_a, add=True)

        @pl.when(c + 2 < nchunk)
        def _():
            _stage(c + 2, m2)
            _stage(c + 3, m3)

        pltpu.make_async_copy(g_hbm.at[sb.at[m1]], rows_b, gsem_b).wait()
        pltpu.make_async_copy(rows_a, accs.at[db.at[m0]], ssem_a).wait()

        @pl.when(c + 2 < nchunk)
        def _():
            _stage_wait(c + 2, m2)
            _stage_wait(c + 3, m3)
            pltpu.async_copy(g_hbm.at[sb.at[m2]], rows_a, gsem_a)

        _scale_chunk(rows_b, wrb, m1)
        pltpu.async_copy(rows_b, accs.at[db.at[m1]], ssem_b, add=True)

    lastm = lax.rem(jnp.int32(nchunk - 1), _RING)
    pltpu.make_async_copy(rows_b, accs.at[db.at[lastm]], ssem_b).wait()
    plsc.subcore_barrier()
    pltpu.sync_copy(accs.at[pl.ds(sid * SL, SL)],
                    out_hbm.at[cid, pl.ds(sid * SL, SL)])


def _sc_aggregate(srcf, dstf, wf, g):
    kern = functools.partial(
        pl.kernel,
        out_type=jax.ShapeDtypeStruct((NC, NP, D), jnp.float32),
        mesh=_mesh,
        scratch_types=[
            pltpu.VMEM((_RING, K), jnp.int32),
            pltpu.VMEM((_RING, K), jnp.int32),
            pltpu.VMEM((_RING, K), jnp.float32),
            pltpu.VMEM((K, D), jnp.float32),
            pltpu.VMEM((K, D), jnp.float32),
            pltpu.VMEM_SHARED((NP, D), jnp.float32),
            pltpu.SemaphoreType.DMA,
            pltpu.SemaphoreType.DMA,
            pltpu.SemaphoreType.DMA,
            pltpu.SemaphoreType.DMA,
            pltpu.SemaphoreType.DMA,
        ],
        compiler_params=_sc_params,
    )(_agg_body)
    znd = jnp.zeros((NP, D), jnp.float32)
    return kern(srcf, dstf, wf, g, znd)


_BN = 400


def _mm_body(x_ref, w_ref, o_ref):
    o_ref[...] = jnp.dot(x_ref[...], w_ref[...],
                         preferred_element_type=jnp.float32,
                         precision=lax.Precision.HIGHEST)


def _tc_matmul(x, W):
    return pl.pallas_call(
        _mm_body,
        grid=(N // _BN,),
        in_specs=[
            pl.BlockSpec((_BN, D), lambda i: (i, 0)),
            pl.BlockSpec((D, D), lambda i: (0, 0)),
        ],
        out_specs=pl.BlockSpec((_BN, D), lambda i: (i, 0)),
        out_shape=jax.ShapeDtypeStruct((N, D), jnp.float32),
    )(x, W)


def _scale_body(degp_ref, h_ref, o_ref):
    deg = degp_ref[:, 0] + degp_ref[:, 1] + 1.0
    dinv = jnp.where(deg > 0, lax.rsqrt(deg), 0.0)
    o_ref[...] = h_ref[...] * dinv[:, None]


def _tc_scale(degp, h):
    return pl.pallas_call(
        _scale_body,
        grid=(N // _BN,),
        in_specs=[
            pl.BlockSpec((_BN, NC), lambda i: (i, 0)),
            pl.BlockSpec((_BN, D), lambda i: (i, 0)),
        ],
        out_specs=pl.BlockSpec((_BN, D), lambda i: (i, 0)),
        out_shape=jax.ShapeDtypeStruct((N, D), jnp.float32),
    )(degp, h)


def _final_body(degp_ref, accp_ref, g_ref, b_ref, o_ref):
    deg = degp_ref[:, 0] + degp_ref[:, 1] + 1.0
    dinv = jnp.where(deg > 0, lax.rsqrt(deg), 0.0)
    acc = accp_ref[0] + accp_ref[1] + g_ref[...]
    o_ref[...] = acc * dinv[:, None] + b_ref[...]


def _tc_final(degp, accp, g, b):
    return pl.pallas_call(
        _final_body,
        grid=(N // _BN,),
        in_specs=[
            pl.BlockSpec((_BN, NC), lambda i: (i, 0)),
            pl.BlockSpec((NC, _BN, D), lambda i: (0, i, 0)),
            pl.BlockSpec((_BN, D), lambda i: (i, 0)),
            pl.BlockSpec((1, D), lambda i: (0, 0)),
        ],
        out_specs=pl.BlockSpec((_BN, D), lambda i: (i, 0)),
        out_shape=jax.ShapeDtypeStruct((N, D), jnp.float32),
    )(degp, accp, g, b)


def kernel(x, edge_index, edge_attr, W, b):
    E = edge_index.shape[1]
    nchunk = -(-E // (NW * K))
    nchunk = -(-nchunk // 4) * 4
    per_tile = nchunk * K
    EP = per_tile * NW
    pad = EP - E

    src = jnp.concatenate([edge_index[0], jnp.zeros((pad,), jnp.int32)])
    dst = jnp.concatenate([edge_index[1], jnp.zeros((pad,), jnp.int32)])
    w = jnp.concatenate([edge_attr, jnp.zeros((pad,), jnp.float32)])
    src3 = src.reshape(NW, nchunk, K)
    dst3 = dst.reshape(NW, nchunk, K)
    w3 = w.reshape(NW, nchunk, K)

    degp = _sc_degree(dst3, w3, nchunk)
    h = _tc_matmul(x, W)
    degp_t = degp[:, :N].T
    g = _tc_scale(degp_t, h)
    accp = _sc_aggregate(src.reshape(NW * nchunk, K),
                         dst.reshape(NW * nchunk, K),
                         w.reshape(NW * nchunk, K), g)
    out = _tc_final(degp_t, accp[:, :N, :], g, b.reshape(1, D))

    out = out.reshape(N, SEQ, D // SEQ)
    out = jnp.transpose(out, (1, 0, 2))
    return out[None]
